# baseline (speedup 1.0000x reference)
import functools

import jax
import jax.numpy as jnp
from jax import lax
from jax.experimental import pallas as pl
from jax.experimental.pallas import tpu as pltpu
from jax.experimental.pallas import tpu_sc as plsc

_N = 1000000
_PAD = 1000448
_BKT = 1024
_NSEP = 17
_L = 16
_NS = 16
_NW = 32
_CH = 4000
_NCH = _N // _CH
_S_OWN = _PAD // _NW
_CB = _S_OWN // 2
_GFULL = 123
_ROWS_PAD = _GFULL * 128
_UN = 10
_UN0 = 5


def _sc_body(hash_hbm, seps_hbm, cx_hbm, cy_hbm, cz_hbm, buf_hbm, pos_hbm,
             seps_v, a_buf, b_buf, winner, idx_v, plane, sem_a, sem_b, sem):
    s = lax.axis_index("s")
    c_ax = lax.axis_index("c")
    wid = c_ax * _NS + s
    iota = lax.iota(jnp.int32, _L)

    pltpu.sync_copy(seps_hbm, seps_v)
    svecs = [seps_v[j] for j in range(_NSEP)]

    nch_mine = jnp.where(s < 10, 16, 15)

    def ph0_chunk(k, _):
        c = s + _NS * k
        e0 = c * _CH
        pltpu.sync_copy(hash_hbm.at[pl.ds(e0, _CH)], a_buf)

        def vec_blk(vi, ibase):
            for u in range(_UN0):
                ivec = ibase + (u * _L)
                h = a_buf[pl.ds(vi * (_UN0 * _L) + u * _L, _L)]
                seg = jnp.zeros((_L,), jnp.int32)
                for j in range(_NSEP):
                    seg = seg + jnp.where(svecs[j] <= ivec, 1, 0)
                p = h + seg * _BKT
                p = p - jnp.where(p >= _PAD, _PAD, 0)
                b_buf[pl.ds(vi * (_UN0 * _L) + u * _L, _L)] = p
            return ibase + _UN0 * _L

        lax.fori_loop(0, _CH // _L // _UN0, vec_blk, iota + e0)
        pltpu.sync_copy(b_buf, pos_hbm.at[pl.ds(e0, _CH)])
        return 0

    lax.fori_loop(0, nch_mine * 0, ph0_chunk, 0)

    neg1 = jnp.full((_L,), -1, jnp.int32)

    def wm(v, _):
        for u in range(8):
            winner[pl.ds(v * (8 * _L) + u * _L, _L)] = neg1
        return 0

    lax.fori_loop(0, _S_OWN // _L // 8, wm, 0)
    winner[pl.ds(_S_OWN - 2 * _L, _L)] = neg1
    winner[pl.ds(_S_OWN - _L, _L)] = neg1

    plsc.subcore_barrier()

    base = wid * _S_OWN
    size_u = jnp.uint32(_S_OWN)

    def scan_vecs(buf, e0):
        def vec_blk(vi, ibase):
            for u in range(_UN):
                p = buf[pl.ds(vi * (_UN * _L) + u * _L, _L)]
                t = p - base
                m = plsc.bitcast(t, jnp.uint32) < size_u
                plsc.store_scatter(winner, [t], ibase + (u * _L), mask=m)
            return ibase + _UN * _L

        lax.fori_loop(0, _CH // _L // _UN, vec_blk, iota + e0)

    pltpu.async_copy(pos_hbm.at[pl.ds(0, _CH)], a_buf, sem_a)

    def scan_pair(c2, _):
        c = 2 * c2
        pltpu.async_copy(pos_hbm.at[pl.ds((c + 1) * _CH, _CH)], b_buf, sem_b)
        pltpu.make_async_copy(pos_hbm.at[pl.ds(0, _CH)], a_buf, sem_a).wait()
        scan_vecs(a_buf, c * _CH)

        @pl.when(c2 < _NCH // 2 - 1)
        def _():
            pltpu.async_copy(pos_hbm.at[pl.ds((c + 2) * _CH, _CH)], a_buf,
                             sem_a)

        pltpu.make_async_copy(pos_hbm.at[pl.ds(0, _CH)], b_buf, sem_b).wait()
        scan_vecs(b_buf, (c + 1) * _CH)
        return 0

    lax.fori_loop(0, _NCH // 2, scan_pair, 0)

    zero = jnp.zeros((_L,), jnp.float32)
    srcs = (cx_hbm, cy_hbm, cz_hbm)
    nv = _CB // _L

    for vv in range(7):
        idx_v[_GFULL - 1, pl.ds(16 + vv * _L, _L)] = iota + (16 + vv * _L)

    for half in range(2):
        off = half * _CB
        r0 = base + off

        def prep(v, kv, off=off):
            wv = winner[pl.ds(off + v * _L, _L)]
            m = wv >= 0
            sf = jnp.where(m, wv, kv)
            idx_v[lax.shift_right_logical(v, 3), pl.ds((v & 7) * _L, _L)] = sf
            return kv + _L

        lax.fori_loop(0, nv, prep, iota)

        for j in range(3):
            def fire(g, _, j=j):
                pltpu.async_copy(srcs[j].at[idx_v.at[g]],
                                 plane.at[pl.ds(g * 128, 128)], sem)
                return 0

            lax.fori_loop(0, _GFULL, fire, 0)
            pltpu.make_async_copy(srcs[j].at[pl.ds(0, _ROWS_PAD)], plane,
                                  sem).wait()

            def zv(v, kl, off=off):
                wv = winner[pl.ds(off + v * _L, _L)]
                mb = wv < 0
                plsc.store_scatter(plane, [kl], zero, mask=mb)
                return kl + _L

            lax.fori_loop(0, nv, zv, iota)
            pltpu.sync_copy(plane.at[pl.ds(0, _CB)],
                            buf_hbm.at[pl.ds(j * _PAD + r0, _CB)])


_sc_scatter = functools.partial(
    pl.kernel,
    out_type=(jax.ShapeDtypeStruct((3 * _PAD,), jnp.float32),
              jax.ShapeDtypeStruct((_N,), jnp.int32)),
    mesh=plsc.VectorSubcoreMesh(core_axis_name="c", subcore_axis_name="s",
                                num_cores=2, num_subcores=_NS),
    scratch_types=[
        pltpu.VMEM((_NSEP, _L), jnp.int32),
        pltpu.VMEM((_CH,), jnp.int32),
        pltpu.VMEM((_CH,), jnp.int32),
        pltpu.VMEM((_S_OWN,), jnp.int32),
        pltpu.VMEM((_GFULL, 128), jnp.int32),
        pltpu.VMEM((_ROWS_PAD,), jnp.float32),
        pltpu.SemaphoreType.DMA,
        pltpu.SemaphoreType.DMA,
        pltpu.SemaphoreType.DMA,
    ],
    compiler_params=pltpu.CompilerParams(needs_layout_passes=False),
)(_sc_body)


_BR = 1024


def _tc_body(buf_ref, wt_ref, b_ref, out_ref):
    feats = buf_ref[...].astype(jnp.bfloat16)
    acc = lax.dot_general(feats, wt_ref[...],
                          dimension_numbers=(((0,), (0,)), ((), ())),
                          preferred_element_type=jnp.float32)
    out_ref[...] = acc.astype(jnp.bfloat16) + b_ref[...]


_tc_embed = pl.pallas_call(
    _tc_body,
    grid=(_PAD // _BR,),
    in_specs=[
        pl.BlockSpec((3, _BR), lambda i: (0, i)),
        pl.BlockSpec((3, 64), lambda i: (0, 0)),
        pl.BlockSpec((1, 64), lambda i: (0, 0)),
    ],
    out_specs=pl.BlockSpec((_BR, 64), lambda i: (i, 0)),
    out_shape=jax.ShapeDtypeStruct((_PAD, 64), jnp.bfloat16),
)


def kernel(coords, seps, hash_idx, W, b):
    seps_b = jnp.broadcast_to(seps.astype(jnp.int32)[:, None], (_NSEP, _L))
    cx = coords[:, 0]
    cy = coords[:, 1]
    cz = coords[:, 2]
    buf, _unused_pos = _sc_scatter(hash_idx, seps_b, cx, cy, cz)
    buf = buf.reshape(3, _PAD)
    wt = W.astype(jnp.bfloat16).T
    bb = b.astype(jnp.bfloat16)[None, :]
    return _tc_embed(buf, wt, bb)


# speedup vs baseline: 1.3984x; 1.3984x over previous
import functools

import jax
import jax.numpy as jnp
from jax import lax
from jax.experimental import pallas as pl
from jax.experimental.pallas import tpu as pltpu
from jax.experimental.pallas import tpu_sc as plsc

_N = 1000000
_PAD = 1000448
_BKT = 1024
_NSEP = 17
_L = 16
_NS = 16
_NW = 32
_CH = 4000
_NCH = _N // _CH
_S_OWN = _PAD // _NW
_CB = _S_OWN // 2
_GFULL = 123
_ROWS_PAD = _GFULL * 128
_UN = 10
_UN0 = 5


def _sc_body(hash_hbm, seps_hbm, cx_hbm, cy_hbm, cz_hbm, buf_hbm, pos_hbm,
             seps_v, a_buf, b_buf, winner, idx_v, plane, sem_a, sem_b, sem):
    s = lax.axis_index("s")
    c_ax = lax.axis_index("c")
    wid = c_ax * _NS + s
    iota = lax.iota(jnp.int32, _L)

    pltpu.sync_copy(seps_hbm, seps_v)
    svecs = [seps_v[j] for j in range(_NSEP)]

    nch_mine = jnp.where(s < 10, 16, 15)

    def ph0_chunk(k, _):
        c = s + _NS * k
        e0 = c * _CH
        pltpu.sync_copy(hash_hbm.at[pl.ds(e0, _CH)], a_buf)

        def vec_blk(vi, ibase):
            for u in range(_UN0):
                ivec = ibase + (u * _L)
                h = a_buf[pl.ds(vi * (_UN0 * _L) + u * _L, _L)]
                seg = jnp.zeros((_L,), jnp.int32)
                for j in range(_NSEP):
                    seg = seg + jnp.where(svecs[j] <= ivec, 1, 0)
                p = h + seg * _BKT
                p = p - jnp.where(p >= _PAD, _PAD, 0)
                b_buf[pl.ds(vi * (_UN0 * _L) + u * _L, _L)] = p
            return ibase + _UN0 * _L

        lax.fori_loop(0, _CH // _L // _UN0, vec_blk, iota + e0)
        pltpu.sync_copy(b_buf, pos_hbm.at[pl.ds(e0, _CH)])
        return 0

    lax.fori_loop(0, nch_mine, ph0_chunk, 0)

    neg1 = jnp.full((_L,), -1, jnp.int32)

    def wm(v, _):
        for u in range(8):
            winner[pl.ds(v * (8 * _L) + u * _L, _L)] = neg1
        return 0

    lax.fori_loop(0, _S_OWN // _L // 8, wm, 0)
    winner[pl.ds(_S_OWN - 2 * _L, _L)] = neg1
    winner[pl.ds(_S_OWN - _L, _L)] = neg1

    plsc.subcore_barrier()

    base = wid * _S_OWN
    size_u = jnp.uint32(_S_OWN)

    def scan_vecs(buf, e0):
        def vec_blk(vi, ibase):
            for u in range(_UN):
                p = buf[pl.ds(vi * (_UN * _L) + u * _L, _L)]
                t = p - base
                m = plsc.bitcast(t, jnp.uint32) < size_u
                plsc.store_scatter(winner, [t], ibase + (u * _L), mask=m)
            return ibase + _UN * _L

        lax.fori_loop(0, _CH // _L // _UN, vec_blk, iota + e0)

    pltpu.async_copy(pos_hbm.at[pl.ds(0, _CH)], a_buf, sem_a)

    def scan_pair(c2, _):
        c = 2 * c2
        pltpu.async_copy(pos_hbm.at[pl.ds((c + 1) * _CH, _CH)], b_buf, sem_b)
        pltpu.make_async_copy(pos_hbm.at[pl.ds(0, _CH)], a_buf, sem_a).wait()
        scan_vecs(a_buf, c * _CH)

        @pl.when(c2 < _NCH // 2 - 1)
        def _():
            pltpu.async_copy(pos_hbm.at[pl.ds((c + 2) * _CH, _CH)], a_buf,
                             sem_a)

        pltpu.make_async_copy(pos_hbm.at[pl.ds(0, _CH)], b_buf, sem_b).wait()
        scan_vecs(b_buf, (c + 1) * _CH)
        return 0

    lax.fori_loop(0, _NCH // 2, scan_pair, 0)

    zero = jnp.zeros((_L,), jnp.float32)
    srcs = (cx_hbm, cy_hbm, cz_hbm)
    nv = _CB // _L

    for vv in range(7):
        idx_v[_GFULL - 1, pl.ds(16 + vv * _L, _L)] = iota + (16 + vv * _L)

    for half in range(2):
        off = half * _CB
        r0 = base + off

        def prep(v, kv, off=off):
            wv = winner[pl.ds(off + v * _L, _L)]
            m = wv >= 0
            sf = jnp.where(m, wv, kv)
            idx_v[lax.shift_right_logical(v, 3), pl.ds((v & 7) * _L, _L)] = sf
            return kv + _L

        lax.fori_loop(0, nv, prep, iota)

        for j in range(3):
            def fire(g, _, j=j):
                pltpu.async_copy(srcs[j].at[idx_v.at[g]],
                                 plane.at[pl.ds(g * 128, 128)], sem)
                return 0

            lax.fori_loop(0, _GFULL, fire, 0)
            pltpu.make_async_copy(srcs[j].at[pl.ds(0, _ROWS_PAD)], plane,
                                  sem).wait()

            def zv(v, kl, off=off):
                wv = winner[pl.ds(off + v * _L, _L)]
                mb = wv < 0
                plsc.store_scatter(plane, [kl], zero, mask=mb)
                return kl + _L

            lax.fori_loop(0, nv, zv, iota)
            pltpu.sync_copy(plane.at[pl.ds(0, _CB)],
                            buf_hbm.at[pl.ds(j * _PAD + r0, _CB)])


_sc_scatter = functools.partial(
    pl.kernel,
    out_type=(jax.ShapeDtypeStruct((3 * _PAD,), jnp.float32),
              jax.ShapeDtypeStruct((_N,), jnp.int32)),
    mesh=plsc.VectorSubcoreMesh(core_axis_name="c", subcore_axis_name="s",
                                num_cores=2, num_subcores=_NS),
    scratch_types=[
        pltpu.VMEM((_NSEP, _L), jnp.int32),
        pltpu.VMEM((_CH,), jnp.int32),
        pltpu.VMEM((_CH,), jnp.int32),
        pltpu.VMEM((_S_OWN,), jnp.int32),
        pltpu.VMEM((_GFULL, 128), jnp.int32),
        pltpu.VMEM((_ROWS_PAD,), jnp.float32),
        pltpu.SemaphoreType.DMA,
        pltpu.SemaphoreType.DMA,
        pltpu.SemaphoreType.DMA,
    ],
    compiler_params=pltpu.CompilerParams(needs_layout_passes=False),
)(_sc_body)


_BR = 8192


def _tc_body(buf_ref, wt_ref, b_ref, out_ref):
    feats = buf_ref[...].astype(jnp.bfloat16)
    acc = lax.dot_general(feats, wt_ref[...],
                          dimension_numbers=(((0,), (0,)), ((), ())),
                          preferred_element_type=jnp.float32)
    out_ref[...] = acc.astype(jnp.bfloat16) + b_ref[...]


_tc_embed = pl.pallas_call(
    _tc_body,
    grid=((_PAD + _BR - 1) // _BR,),
    in_specs=[
        pl.BlockSpec((3, _BR), lambda i: (0, i)),
        pl.BlockSpec((3, 64), lambda i: (0, 0)),
        pl.BlockSpec((1, 64), lambda i: (0, 0)),
    ],
    out_specs=pl.BlockSpec((_BR, 64), lambda i: (i, 0)),
    out_shape=jax.ShapeDtypeStruct((_PAD, 64), jnp.bfloat16),
)


def kernel(coords, seps, hash_idx, W, b):
    seps_b = jnp.broadcast_to(seps.astype(jnp.int32)[:, None], (_NSEP, _L))
    cx = coords[:, 0]
    cy = coords[:, 1]
    cz = coords[:, 2]
    buf, _unused_pos = _sc_scatter(hash_idx, seps_b, cx, cy, cz)
    buf = buf.reshape(3, _PAD)
    wt = W.astype(jnp.bfloat16).T
    bb = b.astype(jnp.bfloat16)[None, :]
    return _tc_embed(buf, wt, bb)


# speedup vs baseline: 1.4620x; 1.0455x over previous
import functools

import jax
import jax.numpy as jnp
from jax import lax
from jax.experimental import pallas as pl
from jax.experimental.pallas import tpu as pltpu
from jax.experimental.pallas import tpu_sc as plsc

_N = 1000000
_PAD = 1000448
_BKT = 1024
_NSEP = 17
_L = 16
_NS = 16
_NW = 32
_CH = 10000
_NCH = _N // _CH
_S_OWN = _PAD // _NW
_CB = _S_OWN // 2
_GFULL = 123
_ROWS_PAD = _GFULL * 128
_UN = 25
_UN0 = 5


def _sc_body(hash_hbm, seps_hbm, cx_hbm, cy_hbm, cz_hbm, buf_hbm, pos_hbm,
             seps_v, a_buf, b_buf, winner, idx_v, plane, sem_a, sem_b, sem):
    s = lax.axis_index("s")
    c_ax = lax.axis_index("c")
    wid = c_ax * _NS + s
    iota = lax.iota(jnp.int32, _L)

    pltpu.sync_copy(seps_hbm, seps_v)
    svecs = [seps_v[j] for j in range(_NSEP)]

    nch_mine = jnp.where(s < 4, 7, 6)

    def ph0_chunk(k, _):
        c = s + _NS * k
        e0 = c * _CH
        pltpu.sync_copy(hash_hbm.at[pl.ds(e0, _CH)], a_buf)

        def vec_blk(vi, ibase):
            for u in range(_UN0):
                ivec = ibase + (u * _L)
                h = a_buf[pl.ds(vi * (_UN0 * _L) + u * _L, _L)]
                seg = jnp.zeros((_L,), jnp.int32)
                for j in range(_NSEP):
                    seg = seg + jnp.where(svecs[j] <= ivec, 1, 0)
                p = h + seg * _BKT
                p = p - jnp.where(p >= _PAD, _PAD, 0)
                b_buf[pl.ds(vi * (_UN0 * _L) + u * _L, _L)] = p
            return ibase + _UN0 * _L

        lax.fori_loop(0, _CH // _L // _UN0, vec_blk, iota + e0)
        pltpu.sync_copy(b_buf, pos_hbm.at[pl.ds(e0, _CH)])
        return 0

    lax.fori_loop(0, nch_mine, ph0_chunk, 0)

    neg1 = jnp.full((_L,), -1, jnp.int32)

    def wm(v, _):
        for u in range(8):
            winner[pl.ds(v * (8 * _L) + u * _L, _L)] = neg1
        return 0

    lax.fori_loop(0, _S_OWN // _L // 8, wm, 0)
    winner[pl.ds(_S_OWN - 2 * _L, _L)] = neg1
    winner[pl.ds(_S_OWN - _L, _L)] = neg1

    plsc.subcore_barrier()

    base = wid * _S_OWN
    size_u = jnp.uint32(_S_OWN)

    def scan_vecs(buf, e0):
        def vec_blk(vi, ibase):
            for u in range(_UN):
                p = buf[pl.ds(vi * (_UN * _L) + u * _L, _L)]
                t = p - base
                m = plsc.bitcast(t, jnp.uint32) < size_u
                plsc.store_scatter(winner, [t], ibase + (u * _L), mask=m)
            return ibase + _UN * _L

        lax.fori_loop(0, _CH // _L // _UN, vec_blk, iota + e0)

    pltpu.async_copy(pos_hbm.at[pl.ds(0, _CH)], a_buf, sem_a)

    def scan_pair(c2, _):
        c = 2 * c2
        pltpu.async_copy(pos_hbm.at[pl.ds((c + 1) * _CH, _CH)], b_buf, sem_b)
        pltpu.make_async_copy(pos_hbm.at[pl.ds(0, _CH)], a_buf, sem_a).wait()
        scan_vecs(a_buf, c * _CH)

        @pl.when(c2 < _NCH // 2 - 1)
        def _():
            pltpu.async_copy(pos_hbm.at[pl.ds((c + 2) * _CH, _CH)], a_buf,
                             sem_a)

        pltpu.make_async_copy(pos_hbm.at[pl.ds(0, _CH)], b_buf, sem_b).wait()
        scan_vecs(b_buf, (c + 1) * _CH)
        return 0

    lax.fori_loop(0, _NCH // 2, scan_pair, 0)

    zero = jnp.zeros((_L,), jnp.float32)
    srcs = (cx_hbm, cy_hbm, cz_hbm)
    nv = _CB // _L

    for vv in range(7):
        idx_v[_GFULL - 1, pl.ds(16 + vv * _L, _L)] = iota + (16 + vv * _L)

    for half in range(2):
        off = half * _CB
        r0 = base + off

        def prep(v, kv, off=off):
            wv = winner[pl.ds(off + v * _L, _L)]
            m = wv >= 0
            sf = jnp.where(m, wv, kv)
            idx_v[lax.shift_right_logical(v, 3), pl.ds((v & 7) * _L, _L)] = sf
            return kv + _L

        lax.fori_loop(0, nv, prep, iota)

        for j in range(3):
            def fire(g, _, j=j):
                pltpu.async_copy(srcs[j].at[idx_v.at[g]],
                                 plane.at[pl.ds(g * 128, 128)], sem)
                return 0

            lax.fori_loop(0, _GFULL, fire, 0)
            pltpu.make_async_copy(srcs[j].at[pl.ds(0, _ROWS_PAD)], plane,
                                  sem).wait()

            def zv(v, kl, off=off):
                wv = winner[pl.ds(off + v * _L, _L)]
                mb = wv < 0
                plsc.store_scatter(plane, [kl], zero, mask=mb)
                return kl + _L

            lax.fori_loop(0, nv, zv, iota)
            pltpu.sync_copy(plane.at[pl.ds(0, _CB)],
                            buf_hbm.at[pl.ds(j * _PAD + r0, _CB)])


_sc_scatter = functools.partial(
    pl.kernel,
    out_type=(jax.ShapeDtypeStruct((3 * _PAD,), jnp.float32),
              jax.ShapeDtypeStruct((_N,), jnp.int32)),
    mesh=plsc.VectorSubcoreMesh(core_axis_name="c", subcore_axis_name="s",
                                num_cores=2, num_subcores=_NS),
    scratch_types=[
        pltpu.VMEM((_NSEP, _L), jnp.int32),
        pltpu.VMEM((_CH,), jnp.int32),
        pltpu.VMEM((_CH,), jnp.int32),
        pltpu.VMEM((_S_OWN,), jnp.int32),
        pltpu.VMEM((_GFULL, 128), jnp.int32),
        pltpu.VMEM((_ROWS_PAD,), jnp.float32),
        pltpu.SemaphoreType.DMA,
        pltpu.SemaphoreType.DMA,
        pltpu.SemaphoreType.DMA,
    ],
    compiler_params=pltpu.CompilerParams(needs_layout_passes=False),
)(_sc_body)


_BR = 32768


def _tc_body(buf_ref, wt_ref, b_ref, out_ref):
    feats = buf_ref[...].astype(jnp.bfloat16)
    acc = lax.dot_general(feats, wt_ref[...],
                          dimension_numbers=(((0,), (0,)), ((), ())),
                          preferred_element_type=jnp.float32)
    out_ref[...] = acc.astype(jnp.bfloat16) + b_ref[...]


_tc_embed = pl.pallas_call(
    _tc_body,
    grid=((_PAD + _BR - 1) // _BR,),
    in_specs=[
        pl.BlockSpec((3, _BR), lambda i: (0, i)),
        pl.BlockSpec((3, 64), lambda i: (0, 0)),
        pl.BlockSpec((1, 64), lambda i: (0, 0)),
    ],
    out_specs=pl.BlockSpec((_BR, 64), lambda i: (i, 0)),
    out_shape=jax.ShapeDtypeStruct((_PAD, 64), jnp.bfloat16),
)


def kernel(coords, seps, hash_idx, W, b):
    seps_b = jnp.broadcast_to(seps.astype(jnp.int32)[:, None], (_NSEP, _L))
    cx = coords[:, 0]
    cy = coords[:, 1]
    cz = coords[:, 2]
    buf, _unused_pos = _sc_scatter(hash_idx, seps_b, cx, cy, cz)
    buf = buf.reshape(3, _PAD)
    wt = W.astype(jnp.bfloat16).T
    bb = b.astype(jnp.bfloat16)[None, :]
    return _tc_embed(buf, wt, bb)


# speedup vs baseline: 1.4634x; 1.0009x over previous
import functools

import jax
import jax.numpy as jnp
from jax import lax
from jax.experimental import pallas as pl
from jax.experimental.pallas import tpu as pltpu
from jax.experimental.pallas import tpu_sc as plsc

_N = 1000000
_PAD = 1000448
_BKT = 1024
_NSEP = 17
_L = 16
_NS = 16
_NW = 32
_CH = 10000
_NCH = _N // _CH
_S_OWN = _PAD // _NW
_CB = _S_OWN // 2
_GFULL = 123
_ROWS_PAD = _GFULL * 128
_UN = 25
_UN0 = 5


def _sc_body(hash_hbm, seps_hbm, cx_hbm, cy_hbm, cz_hbm, buf_hbm, pos_hbm,
             seps_v, a_buf, b_buf, winner, idx_v, plane, sem_a, sem_b, sem):
    s = lax.axis_index("s")
    c_ax = lax.axis_index("c")
    wid = c_ax * _NS + s
    iota = lax.iota(jnp.int32, _L)

    pltpu.sync_copy(seps_hbm, seps_v)
    svecs = [seps_v[j] for j in range(_NSEP)]

    nch_mine = jnp.where(s < 4, 7, 6)

    def ph0_chunk(k, _):
        c = s + _NS * k
        e0 = c * _CH
        pltpu.sync_copy(hash_hbm.at[pl.ds(e0, _CH)], a_buf)

        def vec_blk(vi, ibase):
            for u in range(_UN0):
                ivec = ibase + (u * _L)
                h = a_buf[pl.ds(vi * (_UN0 * _L) + u * _L, _L)]
                seg = jnp.zeros((_L,), jnp.int32)
                for j in range(_NSEP):
                    seg = seg + jnp.where(svecs[j] <= ivec, 1, 0)
                p = h + seg * _BKT
                p = p - jnp.where(p >= _PAD, _PAD, 0)
                b_buf[pl.ds(vi * (_UN0 * _L) + u * _L, _L)] = p
            return ibase + _UN0 * _L

        lax.fori_loop(0, _CH // _L // _UN0, vec_blk, iota + e0)
        pltpu.sync_copy(b_buf, pos_hbm.at[pl.ds(e0, _CH)])
        return 0

    lax.fori_loop(0, nch_mine, ph0_chunk, 0)

    neg1 = jnp.full((_L,), -1, jnp.int32)

    def wm(v, _):
        for u in range(8):
            winner[pl.ds(v * (8 * _L) + u * _L, _L)] = neg1
        return 0

    lax.fori_loop(0, _S_OWN // _L // 8, wm, 0)
    winner[pl.ds(_S_OWN - 2 * _L, _L)] = neg1
    winner[pl.ds(_S_OWN - _L, _L)] = neg1

    plsc.subcore_barrier()

    base = wid * _S_OWN
    size_u = jnp.uint32(_S_OWN)

    def scan_vecs(buf, e0):
        def vec_blk(vi, ibase):
            for u in range(_UN):
                p = buf[pl.ds(vi * (_UN * _L) + u * _L, _L)]
                t = p - base
                m = plsc.bitcast(t, jnp.uint32) < size_u
                plsc.store_scatter(winner, [t], ibase + (u * _L), mask=m)
            return ibase + _UN * _L

        lax.fori_loop(0, _CH // _L // _UN, vec_blk, iota + e0)

    pltpu.async_copy(pos_hbm.at[pl.ds(0, _CH)], a_buf, sem_a)

    def scan_pair(c2, _):
        c = 2 * c2
        pltpu.async_copy(pos_hbm.at[pl.ds((c + 1) * _CH, _CH)], b_buf, sem_b)
        pltpu.make_async_copy(pos_hbm.at[pl.ds(0, _CH)], a_buf, sem_a).wait()
        scan_vecs(a_buf, c * _CH)

        @pl.when(c2 < _NCH // 2 - 1)
        def _():
            pltpu.async_copy(pos_hbm.at[pl.ds((c + 2) * _CH, _CH)], a_buf,
                             sem_a)

        pltpu.make_async_copy(pos_hbm.at[pl.ds(0, _CH)], b_buf, sem_b).wait()
        scan_vecs(b_buf, (c + 1) * _CH)
        return 0

    lax.fori_loop(0, _NCH // 2, scan_pair, 0)

    zero = jnp.zeros((_L,), jnp.float32)
    srcs = (cx_hbm, cy_hbm, cz_hbm)
    nv = _CB // _L

    for vv in range(7):
        idx_v[_GFULL - 1, pl.ds(16 + vv * _L, _L)] = iota + (16 + vv * _L)

    for half in range(2):
        off = half * _CB
        r0 = base + off

        def prep(v, kv, off=off):
            wv = winner[pl.ds(off + v * _L, _L)]
            m = wv >= 0
            sf = jnp.where(m, wv, kv)
            idx_v[lax.shift_right_logical(v, 3), pl.ds((v & 7) * _L, _L)] = sf
            return kv + _L

        lax.fori_loop(0, nv, prep, iota)

        for j in range(3):
            def fire(g, _, j=j):
                pltpu.async_copy(srcs[j].at[idx_v.at[g]],
                                 plane.at[pl.ds(g * 128, 128)], sem)
                return 0

            lax.fori_loop(0, _GFULL, fire, 0)
            pltpu.make_async_copy(srcs[j].at[pl.ds(0, _ROWS_PAD)], plane,
                                  sem).wait()

            def zv(v, kl, off=off):
                wv = winner[pl.ds(off + v * _L, _L)]
                mb = wv < 0
                plsc.store_scatter(plane, [kl], zero, mask=mb)
                return kl + _L

            lax.fori_loop(0, nv, zv, iota)
            pltpu.sync_copy(plane.at[pl.ds(0, _CB)],
                            buf_hbm.at[pl.ds(j * _PAD + r0, _CB)])


_sc_scatter = functools.partial(
    pl.kernel,
    out_type=(jax.ShapeDtypeStruct((3 * _PAD,), jnp.float32),
              jax.ShapeDtypeStruct((_N,), jnp.int32)),
    mesh=plsc.VectorSubcoreMesh(core_axis_name="c", subcore_axis_name="s",
                                num_cores=2, num_subcores=_NS),
    scratch_types=[
        pltpu.VMEM((_NSEP, _L), jnp.int32),
        pltpu.VMEM((_CH,), jnp.int32),
        pltpu.VMEM((_CH,), jnp.int32),
        pltpu.VMEM((_S_OWN,), jnp.int32),
        pltpu.VMEM((_GFULL, 128), jnp.int32),
        pltpu.VMEM((_ROWS_PAD,), jnp.float32),
        pltpu.SemaphoreType.DMA,
        pltpu.SemaphoreType.DMA,
        pltpu.SemaphoreType.DMA,
    ],
    compiler_params=pltpu.CompilerParams(needs_layout_passes=False),
)(_sc_body)


_BR = 32768


def _tc_body(buf_ref, wt_ref, b_ref, out_ref):
    feats = buf_ref[...].astype(jnp.bfloat16)
    acc = lax.dot_general(feats, wt_ref[...],
                          dimension_numbers=(((0,), (0,)), ((), ())),
                          preferred_element_type=jnp.float32)
    out_ref[...] = acc.astype(jnp.bfloat16) + b_ref[...]


_tc_embed = pl.pallas_call(
    _tc_body,
    grid=((_PAD + _BR - 1) // _BR,),
    in_specs=[
        pl.BlockSpec((3, _BR), lambda i: (0, i)),
        pl.BlockSpec((3, 64), lambda i: (0, 0)),
        pl.BlockSpec((1, 64), lambda i: (0, 0)),
    ],
    out_specs=pl.BlockSpec((_BR, 64), lambda i: (i, 0)),
    out_shape=jax.ShapeDtypeStruct((_PAD, 64), jnp.bfloat16),
)


def kernel(coords, seps, hash_idx, W, b):
    seps_b = jnp.broadcast_to(seps.astype(jnp.int32)[:, None], (_NSEP, _L))
    cx = coords[:, 0]
    cy = cx
    cz = cx
    buf, _unused_pos = _sc_scatter(hash_idx, seps_b, cx, cy, cz)
    buf = buf.reshape(3, _PAD)
    wt = W.astype(jnp.bfloat16).T
    bb = b.astype(jnp.bfloat16)[None, :]
    return _tc_embed(buf, wt, bb)


# speedup vs baseline: 2.1445x; 1.4654x over previous
import functools

import jax
import jax.numpy as jnp
from jax import lax
from jax.experimental import pallas as pl
from jax.experimental.pallas import tpu as pltpu
from jax.experimental.pallas import tpu_sc as plsc

_N = 1000000
_PAD = 1000448
_BKT = 1024
_NSEP = 17
_L = 16
_NS = 16
_NW = 32
_CH = 10000
_NCH = _N // _CH
_S_OWN = _PAD // _NW
_CB = _S_OWN // 2
_GFULL = 123
_ROWS_PAD = _GFULL * 128
_UN = 25
_UN0 = 5


def _sc_body(hash_hbm, seps_hbm, cx_hbm, cy_hbm, cz_hbm, buf_hbm, pos_hbm,
             seps_v, a_buf, b_buf, winner, idx_v, plane, sem_a, sem_b, sem):
    s = lax.axis_index("s")
    c_ax = lax.axis_index("c")
    wid = c_ax * _NS + s
    iota = lax.iota(jnp.int32, _L)

    pltpu.sync_copy(seps_hbm, seps_v)
    svecs = [seps_v[j] for j in range(_NSEP)]

    nch_mine = jnp.where(s < 4, 7, 6)

    def ph0_chunk(k, _):
        c = s + _NS * k
        e0 = c * _CH
        pltpu.sync_copy(hash_hbm.at[pl.ds(e0, _CH)], a_buf)

        def vec_blk(vi, ibase):
            for u in range(_UN0):
                ivec = ibase + (u * _L)
                h = a_buf[pl.ds(vi * (_UN0 * _L) + u * _L, _L)]
                seg = jnp.zeros((_L,), jnp.int32)
                for j in range(_NSEP):
                    seg = seg + jnp.where(svecs[j] <= ivec, 1, 0)
                p = h + seg * _BKT
                p = p - jnp.where(p >= _PAD, _PAD, 0)
                b_buf[pl.ds(vi * (_UN0 * _L) + u * _L, _L)] = p
            return ibase + _UN0 * _L

        lax.fori_loop(0, _CH // _L // _UN0, vec_blk, iota + e0)
        pltpu.sync_copy(b_buf, pos_hbm.at[pl.ds(e0, _CH)])
        return 0

    lax.fori_loop(0, nch_mine, ph0_chunk, 0)

    neg1 = jnp.full((_L,), -1, jnp.int32)

    def wm(v, _):
        for u in range(8):
            winner[pl.ds(v * (8 * _L) + u * _L, _L)] = neg1
        return 0

    lax.fori_loop(0, _S_OWN // _L // 8, wm, 0)
    winner[pl.ds(_S_OWN - 2 * _L, _L)] = neg1
    winner[pl.ds(_S_OWN - _L, _L)] = neg1

    plsc.subcore_barrier()

    base = wid * _S_OWN
    size_u = jnp.uint32(_S_OWN)

    def scan_vecs(buf, e0):
        def vec_blk(vi, ibase):
            for u in range(_UN):
                p = buf[pl.ds(vi * (_UN * _L) + u * _L, _L)]
                t = p - base
                m = plsc.bitcast(t, jnp.uint32) < size_u
                plsc.store_scatter(winner, [t], ibase + (u * _L), mask=m)
            return ibase + _UN * _L

        lax.fori_loop(0, _CH // _L // _UN, vec_blk, iota + e0)

    pltpu.async_copy(pos_hbm.at[pl.ds(0, _CH)], a_buf, sem_a)

    def scan_pair(c2, _):
        c = 2 * c2
        pltpu.async_copy(pos_hbm.at[pl.ds((c + 1) * _CH, _CH)], b_buf, sem_b)
        pltpu.make_async_copy(pos_hbm.at[pl.ds(0, _CH)], a_buf, sem_a).wait()
        scan_vecs(a_buf, c * _CH)

        @pl.when(c2 < _NCH // 2 - 1)
        def _():
            pltpu.async_copy(pos_hbm.at[pl.ds((c + 2) * _CH, _CH)], a_buf,
                             sem_a)

        pltpu.make_async_copy(pos_hbm.at[pl.ds(0, _CH)], b_buf, sem_b).wait()
        scan_vecs(b_buf, (c + 1) * _CH)
        return 0

    lax.fori_loop(0, 0 * (_NCH // 2), scan_pair, 0)

    zero = jnp.zeros((_L,), jnp.float32)
    srcs = (cx_hbm, cy_hbm, cz_hbm)
    nv = _CB // _L

    for vv in range(7):
        idx_v[_GFULL - 1, pl.ds(16 + vv * _L, _L)] = iota + (16 + vv * _L)

    for half in range(2):
        off = half * _CB
        r0 = base + off

        def prep(v, kv, off=off):
            wv = winner[pl.ds(off + v * _L, _L)]
            m = wv >= 0
            sf = jnp.where(m, wv, kv)
            idx_v[lax.shift_right_logical(v, 3), pl.ds((v & 7) * _L, _L)] = sf
            return kv + _L

        lax.fori_loop(0, nv, prep, iota)

        for j in range(3):
            def fire(g, _, j=j):
                pltpu.async_copy(srcs[j].at[idx_v.at[g]],
                                 plane.at[pl.ds(g * 128, 128)], sem)
                return 0

            lax.fori_loop(0, _GFULL, fire, 0)
            pltpu.make_async_copy(srcs[j].at[pl.ds(0, _ROWS_PAD)], plane,
                                  sem).wait()

            def zv(v, kl, off=off):
                wv = winner[pl.ds(off + v * _L, _L)]
                mb = wv < 0
                plsc.store_scatter(plane, [kl], zero, mask=mb)
                return kl + _L

            lax.fori_loop(0, nv, zv, iota)
            pltpu.sync_copy(plane.at[pl.ds(0, _CB)],
                            buf_hbm.at[pl.ds(j * _PAD + r0, _CB)])


_sc_scatter = functools.partial(
    pl.kernel,
    out_type=(jax.ShapeDtypeStruct((3 * _PAD,), jnp.float32),
              jax.ShapeDtypeStruct((_N,), jnp.int32)),
    mesh=plsc.VectorSubcoreMesh(core_axis_name="c", subcore_axis_name="s",
                                num_cores=2, num_subcores=_NS),
    scratch_types=[
        pltpu.VMEM((_NSEP, _L), jnp.int32),
        pltpu.VMEM((_CH,), jnp.int32),
        pltpu.VMEM((_CH,), jnp.int32),
        pltpu.VMEM((_S_OWN,), jnp.int32),
        pltpu.VMEM((_GFULL, 128), jnp.int32),
        pltpu.VMEM((_ROWS_PAD,), jnp.float32),
        pltpu.SemaphoreType.DMA,
        pltpu.SemaphoreType.DMA,
        pltpu.SemaphoreType.DMA,
    ],
    compiler_params=pltpu.CompilerParams(needs_layout_passes=False),
)(_sc_body)


_BR = 32768


def _tc_body(buf_ref, wt_ref, b_ref, out_ref):
    feats = buf_ref[...].astype(jnp.bfloat16)
    acc = lax.dot_general(feats, wt_ref[...],
                          dimension_numbers=(((0,), (0,)), ((), ())),
                          preferred_element_type=jnp.float32)
    out_ref[...] = acc.astype(jnp.bfloat16) + b_ref[...]


_tc_embed = pl.pallas_call(
    _tc_body,
    grid=((_PAD + _BR - 1) // _BR,),
    in_specs=[
        pl.BlockSpec((3, _BR), lambda i: (0, i)),
        pl.BlockSpec((3, 64), lambda i: (0, 0)),
        pl.BlockSpec((1, 64), lambda i: (0, 0)),
    ],
    out_specs=pl.BlockSpec((_BR, 64), lambda i: (i, 0)),
    out_shape=jax.ShapeDtypeStruct((_PAD, 64), jnp.bfloat16),
)


def kernel(coords, seps, hash_idx, W, b):
    seps_b = jnp.broadcast_to(seps.astype(jnp.int32)[:, None], (_NSEP, _L))
    cx = coords[:, 0]
    cy = coords[:, 1]
    cz = coords[:, 2]
    buf, _unused_pos = _sc_scatter(hash_idx, seps_b, cx, cy, cz)
    buf = buf.reshape(3, _PAD)
    wt = W.astype(jnp.bfloat16).T
    bb = b.astype(jnp.bfloat16)[None, :]
    return _tc_embed(buf, wt, bb)


# speedup vs baseline: 2.1672x; 1.0106x over previous
import functools

import jax
import jax.numpy as jnp
from jax import lax
from jax.experimental import pallas as pl
from jax.experimental.pallas import tpu as pltpu
from jax.experimental.pallas import tpu_sc as plsc

_N = 1000000
_PAD = 1000448
_BKT = 1024
_NSEP = 17
_L = 16
_NS = 16
_NW = 32
_CH = 10000
_NCH = _N // _CH
_S_OWN = _PAD // _NW
_CB = _S_OWN // 2
_GFULL = 123
_ROWS_PAD = _GFULL * 128
_UN = 25
_UN0 = 5


def _sc_body(hash_hbm, seps_hbm, cx_hbm, cy_hbm, cz_hbm, buf_hbm, pos_hbm,
             seps_v, a_buf, b_buf, winner, idx_v, plane, sem_a, sem_b, sem):
    s = lax.axis_index("s")
    c_ax = lax.axis_index("c")
    wid = c_ax * _NS + s
    iota = lax.iota(jnp.int32, _L)

    pltpu.sync_copy(seps_hbm, seps_v)
    svecs = [seps_v[j] for j in range(_NSEP)]

    nch_mine = jnp.where(s < 4, 7, 6)

    def ph0_chunk(k, _):
        c = s + _NS * k
        e0 = c * _CH
        pltpu.sync_copy(hash_hbm.at[pl.ds(e0, _CH)], a_buf)

        def vec_blk(vi, ibase):
            for u in range(_UN0):
                ivec = ibase + (u * _L)
                h = a_buf[pl.ds(vi * (_UN0 * _L) + u * _L, _L)]
                seg = jnp.zeros((_L,), jnp.int32)
                for j in range(_NSEP):
                    seg = seg + jnp.where(svecs[j] <= ivec, 1, 0)
                p = h + seg * _BKT
                p = p - jnp.where(p >= _PAD, _PAD, 0)
                b_buf[pl.ds(vi * (_UN0 * _L) + u * _L, _L)] = p
            return ibase + _UN0 * _L

        lax.fori_loop(0, _CH // _L // _UN0, vec_blk, iota + e0)
        pltpu.sync_copy(b_buf, pos_hbm.at[pl.ds(e0, _CH)])
        return 0

    lax.fori_loop(0, nch_mine, ph0_chunk, 0)

    neg1 = jnp.full((_L,), -1, jnp.int32)

    def wm(v, _):
        for u in range(8):
            winner[pl.ds(v * (8 * _L) + u * _L, _L)] = neg1
        return 0

    lax.fori_loop(0, _S_OWN // _L // 8, wm, 0)
    winner[pl.ds(_S_OWN - 2 * _L, _L)] = neg1
    winner[pl.ds(_S_OWN - _L, _L)] = neg1

    plsc.subcore_barrier()

    base = wid * _S_OWN
    size_u = jnp.uint32(_S_OWN)

    def scan_vecs(buf, e0):
        def vec_blk(vi, ibase):
            ts = [buf[pl.ds(vi * (_UN * _L) + u * _L, _L)] - base
                  for u in range(_UN)]
            for u in range(_UN):
                m = plsc.bitcast(ts[u], jnp.uint32) < size_u
                plsc.store_scatter(winner, [ts[u]], ibase + (u * _L), mask=m)
            return ibase + _UN * _L

        lax.fori_loop(0, _CH // _L // _UN, vec_blk, iota + e0)

    pltpu.async_copy(pos_hbm.at[pl.ds(0, _CH)], a_buf, sem_a)

    def scan_pair(c2, _):
        c = 2 * c2
        pltpu.async_copy(pos_hbm.at[pl.ds((c + 1) * _CH, _CH)], b_buf, sem_b)
        pltpu.make_async_copy(pos_hbm.at[pl.ds(0, _CH)], a_buf, sem_a).wait()
        scan_vecs(a_buf, c * _CH)

        @pl.when(c2 < _NCH // 2 - 1)
        def _():
            pltpu.async_copy(pos_hbm.at[pl.ds((c + 2) * _CH, _CH)], a_buf,
                             sem_a)

        pltpu.make_async_copy(pos_hbm.at[pl.ds(0, _CH)], b_buf, sem_b).wait()
        scan_vecs(b_buf, (c + 1) * _CH)
        return 0

    lax.fori_loop(0, _NCH // 2, scan_pair, 0)

    zero = jnp.zeros((_L,), jnp.float32)
    srcs = (cx_hbm, cy_hbm, cz_hbm)
    nv = _CB // _L

    for vv in range(7):
        idx_v[_GFULL - 1, pl.ds(16 + vv * _L, _L)] = iota + (16 + vv * _L)

    for half in range(2):
        off = half * _CB
        r0 = base + off

        def prep(v, kv, off=off):
            wv = winner[pl.ds(off + v * _L, _L)]
            m = wv >= 0
            sf = jnp.where(m, wv, kv)
            idx_v[lax.shift_right_logical(v, 3), pl.ds((v & 7) * _L, _L)] = sf
            return kv + _L

        lax.fori_loop(0, nv, prep, iota)

        for j in range(3):
            def fire(g, _, j=j):
                pltpu.async_copy(srcs[j].at[idx_v.at[g]],
                                 plane.at[pl.ds(g * 128, 128)], sem)
                return 0

            lax.fori_loop(0, _GFULL, fire, 0)
            pltpu.make_async_copy(srcs[j].at[pl.ds(0, _ROWS_PAD)], plane,
                                  sem).wait()

            def zv(v, kl, off=off):
                wv = winner[pl.ds(off + v * _L, _L)]
                mb = wv < 0
                plsc.store_scatter(plane, [kl], zero, mask=mb)
                return kl + _L

            lax.fori_loop(0, nv, zv, iota)
            pltpu.sync_copy(plane.at[pl.ds(0, _CB)],
                            buf_hbm.at[pl.ds(j * _PAD + r0, _CB)])


_sc_scatter = functools.partial(
    pl.kernel,
    out_type=(jax.ShapeDtypeStruct((3 * _PAD,), jnp.float32),
              jax.ShapeDtypeStruct((_N,), jnp.int32)),
    mesh=plsc.VectorSubcoreMesh(core_axis_name="c", subcore_axis_name="s",
                                num_cores=2, num_subcores=_NS),
    scratch_types=[
        pltpu.VMEM((_NSEP, _L), jnp.int32),
        pltpu.VMEM((_CH,), jnp.int32),
        pltpu.VMEM((_CH,), jnp.int32),
        pltpu.VMEM((_S_OWN,), jnp.int32),
        pltpu.VMEM((_GFULL, 128), jnp.int32),
        pltpu.VMEM((_ROWS_PAD,), jnp.float32),
        pltpu.SemaphoreType.DMA,
        pltpu.SemaphoreType.DMA,
        pltpu.SemaphoreType.DMA,
    ],
    compiler_params=pltpu.CompilerParams(needs_layout_passes=False),
)(_sc_body)


_BR = 32768


def _tc_body(buf_ref, wt_ref, b_ref, out_ref):
    feats = buf_ref[...].astype(jnp.bfloat16)
    acc = lax.dot_general(feats, wt_ref[...],
                          dimension_numbers=(((0,), (0,)), ((), ())),
                          preferred_element_type=jnp.float32)
    out_ref[...] = acc.astype(jnp.bfloat16) + b_ref[...]


_tc_embed = pl.pallas_call(
    _tc_body,
    grid=((_PAD + _BR - 1) // _BR,),
    in_specs=[
        pl.BlockSpec((3, _BR), lambda i: (0, i)),
        pl.BlockSpec((3, 64), lambda i: (0, 0)),
        pl.BlockSpec((1, 64), lambda i: (0, 0)),
    ],
    out_specs=pl.BlockSpec((_BR, 64), lambda i: (i, 0)),
    out_shape=jax.ShapeDtypeStruct((_PAD, 64), jnp.bfloat16),
)


def kernel(coords, seps, hash_idx, W, b):
    seps_b = jnp.broadcast_to(seps.astype(jnp.int32)[:, None], (_NSEP, _L))
    cx = coords[:, 0]
    cy = coords[:, 1]
    cz = coords[:, 2]
    buf, _unused_pos = _sc_scatter(hash_idx, seps_b, cx, cy, cz)
    buf = buf.reshape(3, _PAD)
    wt = W.astype(jnp.bfloat16).T
    bb = b.astype(jnp.bfloat16)[None, :]
    return _tc_embed(buf, wt, bb)


# speedup vs baseline: 2.3016x; 1.0620x over previous
import functools

import jax
import jax.numpy as jnp
from jax import lax
from jax.experimental import pallas as pl
from jax.experimental.pallas import tpu as pltpu
from jax.experimental.pallas import tpu_sc as plsc

_N = 1000000
_PAD = 1000448
_BKT = 1024
_NSEP = 17
_L = 16
_NS = 16
_NW = 32
_CH = 10000
_NCH = _N // _CH
_S_OWN = _PAD // _NW
_CB = _S_OWN // 2
_GFULL = 123
_ROWS_PAD = _GFULL * 128
_UN = 25
_UN0 = 5


def _sc_body(hash_hbm, seps_hbm, cx_hbm, cy_hbm, cz_hbm, buf_hbm, pos_hbm,
             seps_v, a_buf, b_buf, winner, idx_v, px, py, pz,
             sem_a, sem_b, sem):
    s = lax.axis_index("s")
    c_ax = lax.axis_index("c")
    wid = c_ax * _NS + s
    iota = lax.iota(jnp.int32, _L)

    pltpu.sync_copy(seps_hbm, seps_v)
    svecs = [seps_v[j] for j in range(_NSEP)]

    nch_mine = jnp.where(s < 4, 7, 6)

    def ph0_chunk(k, _):
        c = s + _NS * k
        e0 = c * _CH
        pltpu.sync_copy(hash_hbm.at[pl.ds(e0, _CH)], a_buf)

        def vec_blk(vi, ibase):
            for u in range(_UN0):
                ivec = ibase + (u * _L)
                h = a_buf[pl.ds(vi * (_UN0 * _L) + u * _L, _L)]
                seg = jnp.zeros((_L,), jnp.int32)
                for j in range(_NSEP):
                    seg = seg + jnp.where(svecs[j] <= ivec, 1, 0)
                p = h + seg * _BKT
                p = p - jnp.where(p >= _PAD, _PAD, 0)
                b_buf[pl.ds(vi * (_UN0 * _L) + u * _L, _L)] = p
            return ibase + _UN0 * _L

        lax.fori_loop(0, _CH // _L // _UN0, vec_blk, iota + e0)
        pltpu.sync_copy(b_buf, pos_hbm.at[pl.ds(e0, _CH)])
        return 0

    lax.fori_loop(0, nch_mine, ph0_chunk, 0)

    neg1 = jnp.full((_L,), -1, jnp.int32)

    def wm(v, _):
        for u in range(8):
            winner[pl.ds(v * (8 * _L) + u * _L, _L)] = neg1
        return 0

    lax.fori_loop(0, _S_OWN // _L // 8, wm, 0)
    winner[pl.ds(_S_OWN - 2 * _L, _L)] = neg1
    winner[pl.ds(_S_OWN - _L, _L)] = neg1

    plsc.subcore_barrier()

    base = wid * _S_OWN
    size_u = jnp.uint32(_S_OWN)

    def scan_vecs(buf, e0):
        def vec_blk(vi, ibase):
            ts = [buf[pl.ds(vi * (_UN * _L) + u * _L, _L)] - base
                  for u in range(_UN)]
            for u in range(_UN):
                m = plsc.bitcast(ts[u], jnp.uint32) < size_u
                plsc.store_scatter(winner, [ts[u]], ibase + (u * _L), mask=m)
            return ibase + _UN * _L

        lax.fori_loop(0, _CH // _L // _UN, vec_blk, iota + e0)

    pltpu.async_copy(pos_hbm.at[pl.ds(0, _CH)], a_buf, sem_a)

    def scan_pair(c2, _):
        c = 2 * c2
        pltpu.async_copy(pos_hbm.at[pl.ds((c + 1) * _CH, _CH)], b_buf, sem_b)
        pltpu.make_async_copy(pos_hbm.at[pl.ds(0, _CH)], a_buf, sem_a).wait()
        scan_vecs(a_buf, c * _CH)

        @pl.when(c2 < _NCH // 2 - 1)
        def _():
            pltpu.async_copy(pos_hbm.at[pl.ds((c + 2) * _CH, _CH)], a_buf,
                             sem_a)

        pltpu.make_async_copy(pos_hbm.at[pl.ds(0, _CH)], b_buf, sem_b).wait()
        scan_vecs(b_buf, (c + 1) * _CH)
        return 0

    lax.fori_loop(0, _NCH // 2, scan_pair, 0)

    zero = jnp.zeros((_L,), jnp.float32)
    srcs = (cx_hbm, cy_hbm, cz_hbm)
    nv = _CB // _L

    for vv in range(7):
        idx_v[_GFULL - 1, pl.ds(16 + vv * _L, _L)] = iota + (16 + vv * _L)

    planes = (px, py, pz)
    sems = (sem_a, sem_b, sem)

    for half in range(2):
        off = half * _CB
        r0 = base + off

        def prep_blk(gi, kv, off=off):
            ws = [winner[pl.ds(off + gi * 128 + u * _L, _L)] for u in range(8)]
            for u in range(8):
                m = ws[u] >= 0
                sf = jnp.where(m, ws[u], kv + (u * _L))
                idx_v[gi, pl.ds(u * _L, _L)] = sf
            return kv + 128

        kv_t = lax.fori_loop(0, nv // 8, prep_blk, iota)
        wt_ = winner[pl.ds(off + (nv - 1) * _L, _L)]
        idx_v[_GFULL - 1, pl.ds(0, _L)] = jnp.where(wt_ >= 0, wt_, kv_t)

        for j in range(3):
            def fire(g, _, j=j):
                pltpu.async_copy(srcs[j].at[idx_v.at[g]],
                                 planes[j].at[pl.ds(g * 128, 128)], sems[j])
                return 0

            lax.fori_loop(0, _GFULL, fire, 0)
        for j in range(3):
            pltpu.make_async_copy(srcs[j].at[pl.ds(0, _ROWS_PAD)], planes[j],
                                  sems[j]).wait()

        def zv_blk(gi, kl, off=off):
            ws = [winner[pl.ds(off + gi * 128 + u * _L, _L)] for u in range(8)]
            for u in range(8):
                mb = ws[u] < 0
                for j in range(3):
                    plsc.store_scatter(planes[j], [kl + (u * _L)], zero,
                                       mask=mb)
            return kl + 128

        kl_t = lax.fori_loop(0, nv // 8, zv_blk, iota)
        mb_t = winner[pl.ds(off + (nv - 1) * _L, _L)] < 0
        for j in range(3):
            plsc.store_scatter(planes[j], [kl_t], zero, mask=mb_t)
            pltpu.sync_copy(planes[j].at[pl.ds(0, _CB)],
                            buf_hbm.at[pl.ds(j * _PAD + r0, _CB)])


_sc_scatter = functools.partial(
    pl.kernel,
    out_type=(jax.ShapeDtypeStruct((3 * _PAD,), jnp.float32),
              jax.ShapeDtypeStruct((_N,), jnp.int32)),
    mesh=plsc.VectorSubcoreMesh(core_axis_name="c", subcore_axis_name="s",
                                num_cores=2, num_subcores=_NS),
    scratch_types=[
        pltpu.VMEM((_NSEP, _L), jnp.int32),
        pltpu.VMEM((_CH,), jnp.int32),
        pltpu.VMEM((_CH,), jnp.int32),
        pltpu.VMEM((_S_OWN,), jnp.int32),
        pltpu.VMEM((_GFULL, 128), jnp.int32),
        pltpu.VMEM((_ROWS_PAD,), jnp.float32),
        pltpu.VMEM((_ROWS_PAD,), jnp.float32),
        pltpu.VMEM((_ROWS_PAD,), jnp.float32),
        pltpu.SemaphoreType.DMA,
        pltpu.SemaphoreType.DMA,
        pltpu.SemaphoreType.DMA,
    ],
    compiler_params=pltpu.CompilerParams(needs_layout_passes=False),
)(_sc_body)


_BR = 32768


def _tc_body(buf_ref, wt_ref, b_ref, out_ref):
    feats = buf_ref[...].astype(jnp.bfloat16)
    acc = lax.dot_general(feats, wt_ref[...],
                          dimension_numbers=(((0,), (0,)), ((), ())),
                          preferred_element_type=jnp.float32)
    out_ref[...] = acc.astype(jnp.bfloat16) + b_ref[...]


_tc_embed = pl.pallas_call(
    _tc_body,
    grid=((_PAD + _BR - 1) // _BR,),
    in_specs=[
        pl.BlockSpec((3, _BR), lambda i: (0, i)),
        pl.BlockSpec((3, 64), lambda i: (0, 0)),
        pl.BlockSpec((1, 64), lambda i: (0, 0)),
    ],
    out_specs=pl.BlockSpec((_BR, 64), lambda i: (i, 0)),
    out_shape=jax.ShapeDtypeStruct((_PAD, 64), jnp.bfloat16),
)


def kernel(coords, seps, hash_idx, W, b):
    seps_b = jnp.broadcast_to(seps.astype(jnp.int32)[:, None], (_NSEP, _L))
    cx = coords[:, 0]
    cy = coords[:, 1]
    cz = coords[:, 2]
    buf, _unused_pos = _sc_scatter(hash_idx, seps_b, cx, cy, cz)
    buf = buf.reshape(3, _PAD)
    wt = W.astype(jnp.bfloat16).T
    bb = b.astype(jnp.bfloat16)[None, :]
    return _tc_embed(buf, wt, bb)


# speedup vs baseline: 2.3031x; 1.0006x over previous
import functools

import jax
import jax.numpy as jnp
from jax import lax
from jax.experimental import pallas as pl
from jax.experimental.pallas import tpu as pltpu
from jax.experimental.pallas import tpu_sc as plsc

_N = 1000000
_PAD = 1000448
_BKT = 1024
_NSEP = 17
_L = 16
_NS = 16
_NW = 32
_CH = 10000
_NCH = _N // _CH
_S_OWN = _PAD // _NW
_CB = _S_OWN // 2
_GFULL = 123
_ROWS_PAD = _GFULL * 128
_UN = 25
_UN0 = 5


def _sc_body(hash_hbm, seps_hbm, cx_hbm, cy_hbm, cz_hbm, buf_hbm, pos_hbm,
             seps_v, a_buf, b_buf, winner, idx_v, px, py, pz,
             sem_a, sem_b, sem):
    s = lax.axis_index("s")
    c_ax = lax.axis_index("c")
    wid = c_ax * _NS + s
    iota = lax.iota(jnp.int32, _L)

    pltpu.sync_copy(seps_hbm, seps_v)
    svecs = [seps_v[j] for j in range(_NSEP)]

    nch_mine = jnp.where(s < 4, 7, 6)

    def ph0_chunk(k, _):
        c = s + _NS * k
        e0 = c * _CH
        pltpu.sync_copy(hash_hbm.at[pl.ds(e0, _CH)], a_buf)

        def vec_blk(vi, ibase):
            hs = [a_buf[pl.ds(vi * (_UN0 * _L) + u * _L, _L)]
                  for u in range(_UN0)]
            for u in range(_UN0):
                ivec = ibase + (u * _L)
                seg = jnp.zeros((_L,), jnp.int32)
                for j in range(_NSEP):
                    seg = seg + jnp.where(svecs[j] <= ivec, 1, 0)
                p = hs[u] + seg * _BKT
                p = p - jnp.where(p >= _PAD, _PAD, 0)
                b_buf[pl.ds(vi * (_UN0 * _L) + u * _L, _L)] = p
            return ibase + _UN0 * _L

        lax.fori_loop(0, _CH // _L // _UN0, vec_blk, iota + e0)
        pltpu.sync_copy(b_buf, pos_hbm.at[pl.ds(e0, _CH)])
        return 0

    lax.fori_loop(0, nch_mine, ph0_chunk, 0)

    neg1 = jnp.full((_L,), -1, jnp.int32)

    def wm(v, _):
        for u in range(8):
            winner[pl.ds(v * (8 * _L) + u * _L, _L)] = neg1
        return 0

    lax.fori_loop(0, _S_OWN // _L // 8, wm, 0)
    winner[pl.ds(_S_OWN - 2 * _L, _L)] = neg1
    winner[pl.ds(_S_OWN - _L, _L)] = neg1

    plsc.subcore_barrier()

    base = wid * _S_OWN
    size_u = jnp.uint32(_S_OWN)

    def scan_vecs(buf, e0):
        def vec_blk(vi, ibase):
            ts = [buf[pl.ds(vi * (_UN * _L) + u * _L, _L)] - base
                  for u in range(_UN)]
            for u in range(_UN):
                m = plsc.bitcast(ts[u], jnp.uint32) < size_u
                plsc.store_scatter(winner, [ts[u]], ibase + (u * _L), mask=m)
            return ibase + _UN * _L

        lax.fori_loop(0, _CH // _L // _UN, vec_blk, iota + e0)

    pltpu.async_copy(pos_hbm.at[pl.ds(0, _CH)], a_buf, sem_a)

    def scan_pair(c2, _):
        c = 2 * c2
        pltpu.async_copy(pos_hbm.at[pl.ds((c + 1) * _CH, _CH)], b_buf, sem_b)
        pltpu.make_async_copy(pos_hbm.at[pl.ds(0, _CH)], a_buf, sem_a).wait()
        scan_vecs(a_buf, c * _CH)

        @pl.when(c2 < _NCH // 2 - 1)
        def _():
            pltpu.async_copy(pos_hbm.at[pl.ds((c + 2) * _CH, _CH)], a_buf,
                             sem_a)

        pltpu.make_async_copy(pos_hbm.at[pl.ds(0, _CH)], b_buf, sem_b).wait()
        scan_vecs(b_buf, (c + 1) * _CH)
        return 0

    lax.fori_loop(0, _NCH // 2, scan_pair, 0)

    zero = jnp.zeros((_L,), jnp.float32)
    srcs = (cx_hbm, cy_hbm, cz_hbm)
    nv = _CB // _L

    for vv in range(7):
        idx_v[_GFULL - 1, pl.ds(16 + vv * _L, _L)] = iota + (16 + vv * _L)

    planes = (px, py, pz)
    sems = (sem_a, sem_b, sem)

    for half in range(2):
        off = half * _CB
        r0 = base + off

        def prep_blk(gi, kv, off=off):
            ws = [winner[pl.ds(off + gi * 128 + u * _L, _L)] for u in range(8)]
            for u in range(8):
                m = ws[u] >= 0
                sf = jnp.where(m, ws[u], kv + (u * _L))
                idx_v[gi, pl.ds(u * _L, _L)] = sf
            return kv + 128

        kv_t = lax.fori_loop(0, nv // 8, prep_blk, iota)
        wt_ = winner[pl.ds(off + (nv - 1) * _L, _L)]
        idx_v[_GFULL - 1, pl.ds(0, _L)] = jnp.where(wt_ >= 0, wt_, kv_t)

        for j in range(3):
            def fire(g, _, j=j):
                pltpu.async_copy(srcs[j].at[idx_v.at[g]],
                                 planes[j].at[pl.ds(g * 128, 128)], sems[j])
                return 0

            lax.fori_loop(0, _GFULL, fire, 0)
        for j in range(3):
            pltpu.make_async_copy(srcs[j].at[pl.ds(0, _ROWS_PAD)], planes[j],
                                  sems[j]).wait()

        def zv_blk(gi, kl, off=off):
            ws = [winner[pl.ds(off + gi * 128 + u * _L, _L)] for u in range(8)]
            for u in range(8):
                mb = ws[u] < 0
                for j in range(3):
                    plsc.store_scatter(planes[j], [kl + (u * _L)], zero,
                                       mask=mb)
            return kl + 128

        kl_t = lax.fori_loop(0, nv // 8, zv_blk, iota)
        mb_t = winner[pl.ds(off + (nv - 1) * _L, _L)] < 0
        for j in range(3):
            plsc.store_scatter(planes[j], [kl_t], zero, mask=mb_t)
            pltpu.sync_copy(planes[j].at[pl.ds(0, _CB)],
                            buf_hbm.at[pl.ds(j * _PAD + r0, _CB)])


_sc_scatter = functools.partial(
    pl.kernel,
    out_type=(jax.ShapeDtypeStruct((3 * _PAD,), jnp.float32),
              jax.ShapeDtypeStruct((_N,), jnp.int32)),
    mesh=plsc.VectorSubcoreMesh(core_axis_name="c", subcore_axis_name="s",
                                num_cores=2, num_subcores=_NS),
    scratch_types=[
        pltpu.VMEM((_NSEP, _L), jnp.int32),
        pltpu.VMEM((_CH,), jnp.int32),
        pltpu.VMEM((_CH,), jnp.int32),
        pltpu.VMEM((_S_OWN,), jnp.int32),
        pltpu.VMEM((_GFULL, 128), jnp.int32),
        pltpu.VMEM((_ROWS_PAD,), jnp.float32),
        pltpu.VMEM((_ROWS_PAD,), jnp.float32),
        pltpu.VMEM((_ROWS_PAD,), jnp.float32),
        pltpu.SemaphoreType.DMA,
        pltpu.SemaphoreType.DMA,
        pltpu.SemaphoreType.DMA,
    ],
    compiler_params=pltpu.CompilerParams(needs_layout_passes=False),
)(_sc_body)


_BR = 32768


def _tc_body(buf_ref, wt_ref, b_ref, out_ref):
    feats = buf_ref[...].astype(jnp.bfloat16)
    acc = lax.dot_general(feats, wt_ref[...],
                          dimension_numbers=(((0,), (0,)), ((), ())),
                          preferred_element_type=jnp.float32)
    out_ref[...] = acc.astype(jnp.bfloat16) + b_ref[...]


_tc_embed = pl.pallas_call(
    _tc_body,
    grid=((_PAD + _BR - 1) // _BR,),
    in_specs=[
        pl.BlockSpec((3, _BR), lambda i: (0, i)),
        pl.BlockSpec((3, 64), lambda i: (0, 0)),
        pl.BlockSpec((1, 64), lambda i: (0, 0)),
    ],
    out_specs=pl.BlockSpec((_BR, 64), lambda i: (i, 0)),
    out_shape=jax.ShapeDtypeStruct((_PAD, 64), jnp.bfloat16),
)


def kernel(coords, seps, hash_idx, W, b):
    seps_b = jnp.broadcast_to(seps.astype(jnp.int32)[:, None], (_NSEP, _L))
    cx = coords[:, 0]
    cy = coords[:, 1]
    cz = coords[:, 2]
    buf, _unused_pos = _sc_scatter(hash_idx, seps_b, cx, cy, cz)
    buf = buf.reshape(3, _PAD)
    wt = W.astype(jnp.bfloat16).T
    bb = b.astype(jnp.bfloat16)[None, :]
    return _tc_embed(buf, wt, bb)
